# baseline (speedup 1.0000x reference)
import functools

import jax
import jax.numpy as jnp
from jax import lax
from jax.experimental import pallas as pl
from jax.experimental.pallas import tpu as pltpu
from jax.experimental.pallas import tpu_sc as plsc

N_NODES = 10000
N_EDGES = 320000
D = 128

NC = 2
NS = 16
NW = NC * NS
E_PER_W = N_EDGES // NW
CHUNK = 80
NCHUNK = E_PER_W // CHUNK
NROUND = (NCHUNK - 2) // 8
RING_G = 4
RING_I = 8
DS = RING_G // 2
N_PAD = 10240
ROWS_PER_TILE = N_PAD // NS


def _sc_aggregate_body(ei_hbm, sup_hbm, zeros_hbm, out_hbm,
                       ibuf, gbuf, agg_sh, zsem, isem, gsem, ssem):
    cid = lax.axis_index("c")
    sid = lax.axis_index("s")
    wid = sid * NC + cid

    def fire_idx(c, q):
        pltpu.async_copy(ei_hbm.at[wid, c, 0], ibuf.at[q, 0], isem.at[q])
        pltpu.async_copy(ei_hbm.at[wid, c, 1], ibuf.at[q, 1], isem.at[q])

    def wait_idx(q):
        pltpu.make_async_copy(ei_hbm.at[wid, 0, 0], ibuf.at[q, 0],
                              isem.at[q]).wait()
        pltpu.make_async_copy(ei_hbm.at[wid, 0, 1], ibuf.at[q, 1],
                              isem.at[q]).wait()

    def fire_gather(q, g):
        pltpu.async_copy(sup_hbm.at[ibuf.at[q, 1]], gbuf.at[g], gsem.at[g])

    def wait_gather(g):
        pltpu.make_async_copy(sup_hbm.at[pl.ds(0, CHUNK)], gbuf.at[g],
                              gsem.at[g]).wait()

    def fire_scatter(q, g):
        pltpu.async_copy(gbuf.at[g], agg_sh.at[ibuf.at[q, 0]],
                         ssem.at[g], add=True)

    def wait_scatter(g):
        pltpu.make_async_copy(gbuf.at[g], agg_sh.at[pl.ds(0, CHUNK)],
                              ssem.at[g]).wait()

    tile_base = pl.multiple_of(sid * ROWS_PER_TILE, 8)
    iz = pltpu.async_copy(zeros_hbm,
                          agg_sh.at[pl.ds(tile_base, ROWS_PER_TILE)], zsem)
    fire_idx(0, 0)
    fire_idx(1, 1)
    iz.wait()
    plsc.subcore_barrier()

    @pl.loop(0, NROUND)
    def _round(r):
        for j in range(8):
            c = r * 8 + j

            def _wait_s(j=j):
                wait_scatter((j - DS - 1) % RING_G)

            def _fire_i(c=c, j=j):
                fire_idx(c + 2, (j + 2) % RING_I)

            def _ws_fs(j=j):
                wait_gather((j - DS) % RING_G)
                fire_scatter((j - DS) % RING_I, (j - DS) % RING_G)

            if j < DS + 1:
                pl.when(r > 0)(_wait_s)
            else:
                _wait_s()
            _fire_i()
            wait_idx(j)
            fire_gather(j, j % RING_G)
            if j < DS:
                pl.when(r > 0)(_ws_fs)
            else:
                _ws_fs()

    for c in range(NROUND * 8, NCHUNK):
        wait_scatter((c - DS - 1) % RING_G)
        if c + 2 < NCHUNK:
            fire_idx(c + 2, (c + 2) % RING_I)
        wait_idx(c % RING_I)
        fire_gather(c % RING_I, c % RING_G)
        wait_gather((c - DS) % RING_G)
        fire_scatter((c - DS) % RING_I, (c - DS) % RING_G)
    for k in range(NCHUNK - DS, NCHUNK):
        wait_scatter((k - 1) % RING_G)
        wait_gather(k % RING_G)
        fire_scatter(k % RING_I, k % RING_G)
    wait_scatter((NCHUNK - 1) % RING_G)

    plsc.subcore_barrier()

    @pl.when(sid < NS - 1)
    def _copy_full():
        pltpu.sync_copy(agg_sh.at[pl.ds(tile_base, ROWS_PER_TILE)],
                        out_hbm.at[cid, pl.ds(tile_base, ROWS_PER_TILE)])

    @pl.when(sid == NS - 1)
    def _copy_tail():
        tail = N_NODES - (NS - 1) * ROWS_PER_TILE
        pltpu.sync_copy(agg_sh.at[pl.ds(tile_base, tail)],
                        out_hbm.at[cid, pl.ds(tile_base, tail)])


@functools.cache
def _sc_aggregate():
    mesh = plsc.VectorSubcoreMesh(core_axis_name="c", subcore_axis_name="s",
                                  num_cores=NC, num_subcores=NS)
    return pl.kernel(
        _sc_aggregate_body,
        out_type=jax.ShapeDtypeStruct((NC, N_NODES, D), jnp.float32),
        mesh=mesh,
        scratch_types=[
            pltpu.VMEM((RING_I, 2, CHUNK), jnp.int32),
            pltpu.VMEM((RING_G, CHUNK, D), jnp.float32),
            pltpu.VMEM_SHARED((N_PAD, D), jnp.float32),
            pltpu.SemaphoreType.DMA,
            pltpu.SemaphoreType.DMA((RING_I,)),
            pltpu.SemaphoreType.DMA((RING_G,)),
            pltpu.SemaphoreType.DMA((RING_G,)),
        ],
    )


_BLK = 10000


def _mm_body(x_ref, w_ref, o_ref):
    o_ref[...] = jnp.dot(x_ref[...], w_ref[...],
                         preferred_element_type=jnp.float32)


def _tc_matmul(x, w):
    return pl.pallas_call(
        _mm_body,
        grid=(N_NODES // _BLK,),
        in_specs=[pl.BlockSpec((_BLK, D), lambda i: (i, 0)),
                  pl.BlockSpec((D, D), lambda i: (0, 0))],
        out_specs=pl.BlockSpec((_BLK, D), lambda i: (i, 0)),
        out_shape=jax.ShapeDtypeStruct((N_NODES, D), jnp.float32),
    )(x, w)


def _relu_mm_body(p_ref, w_ref, o_ref):
    x = jnp.maximum(p_ref[0] + p_ref[1], 0.0)
    o_ref[...] = jnp.dot(x, w_ref[...], preferred_element_type=jnp.float32)


def _tc_relu_matmul(p, w):
    return pl.pallas_call(
        _relu_mm_body,
        grid=(N_NODES // _BLK,),
        in_specs=[pl.BlockSpec((NC, _BLK, D), lambda i: (0, i, 0)),
                  pl.BlockSpec((D, D), lambda i: (0, 0))],
        out_specs=pl.BlockSpec((_BLK, D), lambda i: (i, 0)),
        out_shape=jax.ShapeDtypeStruct((N_NODES, D), jnp.float32),
    )(p, w)


def _relu_mm_bias_body(p_ref, w_ref, b_ref, o_ref):
    x = jnp.maximum(p_ref[0] + p_ref[1], 0.0)
    o_ref[...] = (jnp.dot(x, w_ref[...], preferred_element_type=jnp.float32)
                  + b_ref[...])


def _tc_relu_matmul_bias(p, w, b):
    return pl.pallas_call(
        _relu_mm_bias_body,
        grid=(N_NODES // _BLK,),
        in_specs=[pl.BlockSpec((NC, _BLK, D), lambda i: (0, i, 0)),
                  pl.BlockSpec((D, D), lambda i: (0, 0)),
                  pl.BlockSpec((1, D), lambda i: (0, 0))],
        out_specs=pl.BlockSpec((_BLK, D), lambda i: (i, 0)),
        out_shape=jax.ShapeDtypeStruct((N_NODES, D), jnp.float32),
    )(p, w, b.reshape(1, D))


def kernel(edge_index, features, W1, W2, Wout, bout):
    ei = edge_index.reshape(2, NW, NCHUNK, CHUNK).transpose(1, 2, 0, 3)
    zeros = jnp.zeros((ROWS_PER_TILE, D), jnp.float32)
    support1 = _tc_matmul(features, W1)
    agg = _sc_aggregate()
    p1 = agg(ei, support1, zeros)
    support2 = _tc_relu_matmul(p1, W2)
    p2 = agg(ei, support2, zeros)
    return _tc_relu_matmul_bias(p2, Wout, bout)


# speedup vs baseline: 1.0479x; 1.0479x over previous
import functools

import jax
import jax.numpy as jnp
from jax import lax
from jax.experimental import pallas as pl
from jax.experimental.pallas import tpu as pltpu
from jax.experimental.pallas import tpu_sc as plsc

N_NODES = 10000
N_EDGES = 320000
D = 128

NC = 2
NS = 16
NW = NC * NS
E_PER_W = N_EDGES // NW
CHUNK = 80
NCHUNK = E_PER_W // CHUNK
NROUND = (NCHUNK - 2) // 8
RING_G = 4
RING_I = 8
DS = RING_G - 1
N_PAD = 10240
ROWS_PER_TILE = N_PAD // NS


def _sc_aggregate_body(ei_hbm, sup_hbm, zeros_hbm, out_hbm,
                       ibuf, gbuf, agg_sh, zsem, isem, gsem, ssem):
    cid = lax.axis_index("c")
    sid = lax.axis_index("s")
    wid = sid * NC + cid

    def fire_idx(c, q):
        pltpu.async_copy(ei_hbm.at[wid, c, 0], ibuf.at[q, 0], isem.at[q])
        pltpu.async_copy(ei_hbm.at[wid, c, 1], ibuf.at[q, 1], isem.at[q])

    def wait_idx(q):
        pltpu.make_async_copy(ei_hbm.at[wid, 0, 0], ibuf.at[q, 0],
                              isem.at[q]).wait()
        pltpu.make_async_copy(ei_hbm.at[wid, 0, 1], ibuf.at[q, 1],
                              isem.at[q]).wait()

    def fire_gather(q, g):
        pltpu.async_copy(sup_hbm.at[ibuf.at[q, 1]], gbuf.at[g], gsem.at[g])

    def wait_gather(g):
        pltpu.make_async_copy(sup_hbm.at[pl.ds(0, CHUNK)], gbuf.at[g],
                              gsem.at[g]).wait()

    def fire_scatter(q, g):
        pltpu.async_copy(gbuf.at[g], agg_sh.at[ibuf.at[q, 0]],
                         ssem.at[g], add=True)

    def wait_scatter(g):
        pltpu.make_async_copy(gbuf.at[g], agg_sh.at[pl.ds(0, CHUNK)],
                              ssem.at[g]).wait()

    tile_base = pl.multiple_of(sid * ROWS_PER_TILE, 8)
    iz = pltpu.async_copy(zeros_hbm,
                          agg_sh.at[pl.ds(tile_base, ROWS_PER_TILE)], zsem)
    fire_idx(0, 0)
    fire_idx(1, 1)
    iz.wait()
    plsc.subcore_barrier()

    @pl.loop(0, NROUND)
    def _round(r):
        for j in range(8):
            c = r * 8 + j

            def _wait_s(j=j):
                wait_scatter((j - DS - 1) % RING_G)

            def _fire_i(c=c, j=j):
                fire_idx(c + 2, (j + 2) % RING_I)

            def _ws_fs(j=j):
                wait_gather((j - DS) % RING_G)
                fire_scatter((j - DS) % RING_I, (j - DS) % RING_G)

            if j < DS + 1:
                pl.when(r > 0)(_wait_s)
            else:
                _wait_s()
            _fire_i()
            wait_idx(j)
            fire_gather(j, j % RING_G)
            if j < DS:
                pl.when(r > 0)(_ws_fs)
            else:
                _ws_fs()

    for c in range(NROUND * 8, NCHUNK):
        wait_scatter((c - DS - 1) % RING_G)
        if c + 2 < NCHUNK:
            fire_idx(c + 2, (c + 2) % RING_I)
        wait_idx(c % RING_I)
        fire_gather(c % RING_I, c % RING_G)
        wait_gather((c - DS) % RING_G)
        fire_scatter((c - DS) % RING_I, (c - DS) % RING_G)
    for k in range(NCHUNK - DS, NCHUNK):
        wait_scatter((k - 1) % RING_G)
        wait_gather(k % RING_G)
        fire_scatter(k % RING_I, k % RING_G)
    wait_scatter((NCHUNK - 1) % RING_G)

    plsc.subcore_barrier()

    @pl.when(sid < NS - 1)
    def _copy_full():
        pltpu.sync_copy(agg_sh.at[pl.ds(tile_base, ROWS_PER_TILE)],
                        out_hbm.at[cid, pl.ds(tile_base, ROWS_PER_TILE)])

    @pl.when(sid == NS - 1)
    def _copy_tail():
        tail = N_NODES - (NS - 1) * ROWS_PER_TILE
        pltpu.sync_copy(agg_sh.at[pl.ds(tile_base, tail)],
                        out_hbm.at[cid, pl.ds(tile_base, tail)])


@functools.cache
def _sc_aggregate():
    mesh = plsc.VectorSubcoreMesh(core_axis_name="c", subcore_axis_name="s",
                                  num_cores=NC, num_subcores=NS)
    return pl.kernel(
        _sc_aggregate_body,
        out_type=jax.ShapeDtypeStruct((NC, N_NODES, D), jnp.float32),
        mesh=mesh,
        scratch_types=[
            pltpu.VMEM((RING_I, 2, CHUNK), jnp.int32),
            pltpu.VMEM((RING_G, CHUNK, D), jnp.float32),
            pltpu.VMEM_SHARED((N_PAD, D), jnp.float32),
            pltpu.SemaphoreType.DMA,
            pltpu.SemaphoreType.DMA((RING_I,)),
            pltpu.SemaphoreType.DMA((RING_G,)),
            pltpu.SemaphoreType.DMA((RING_G,)),
        ],
    )


_BLK = 5000


def _mm_body(x_ref, w_ref, o_ref):
    o_ref[...] = jnp.dot(x_ref[...], w_ref[...],
                         preferred_element_type=jnp.float32)


def _tc_matmul(x, w):
    return pl.pallas_call(
        _mm_body,
        grid=(N_NODES // _BLK,),
        in_specs=[pl.BlockSpec((_BLK, D), lambda i: (i, 0)),
                  pl.BlockSpec((D, D), lambda i: (0, 0))],
        out_specs=pl.BlockSpec((_BLK, D), lambda i: (i, 0)),
        out_shape=jax.ShapeDtypeStruct((N_NODES, D), jnp.float32),
    )(x, w)


def _relu_mm_body(p_ref, w_ref, o_ref):
    x = jnp.maximum(p_ref[0] + p_ref[1], 0.0)
    o_ref[...] = jnp.dot(x, w_ref[...], preferred_element_type=jnp.float32)


def _tc_relu_matmul(p, w):
    return pl.pallas_call(
        _relu_mm_body,
        grid=(N_NODES // _BLK,),
        in_specs=[pl.BlockSpec((NC, _BLK, D), lambda i: (0, i, 0)),
                  pl.BlockSpec((D, D), lambda i: (0, 0))],
        out_specs=pl.BlockSpec((_BLK, D), lambda i: (i, 0)),
        out_shape=jax.ShapeDtypeStruct((N_NODES, D), jnp.float32),
    )(p, w)


def _relu_mm_bias_body(p_ref, w_ref, b_ref, o_ref):
    x = jnp.maximum(p_ref[0] + p_ref[1], 0.0)
    o_ref[...] = (jnp.dot(x, w_ref[...], preferred_element_type=jnp.float32)
                  + b_ref[...])


def _tc_relu_matmul_bias(p, w, b):
    return pl.pallas_call(
        _relu_mm_bias_body,
        grid=(N_NODES // _BLK,),
        in_specs=[pl.BlockSpec((NC, _BLK, D), lambda i: (0, i, 0)),
                  pl.BlockSpec((D, D), lambda i: (0, 0)),
                  pl.BlockSpec((1, D), lambda i: (0, 0))],
        out_specs=pl.BlockSpec((_BLK, D), lambda i: (i, 0)),
        out_shape=jax.ShapeDtypeStruct((N_NODES, D), jnp.float32),
    )(p, w, b.reshape(1, D))


def kernel(edge_index, features, W1, W2, Wout, bout):
    ei = edge_index.reshape(2, NW, NCHUNK, CHUNK).transpose(1, 2, 0, 3)
    zeros = jnp.zeros((ROWS_PER_TILE, D), jnp.float32)
    support1 = _tc_matmul(features, W1)
    agg = _sc_aggregate()
    p1 = agg(ei, support1, zeros)
    support2 = _tc_relu_matmul(p1, W2)
    p2 = agg(ei, support2, zeros)
    return _tc_relu_matmul_bias(p2, Wout, bout)
